# Initial kernel scaffold; baseline (speedup 1.0000x reference)
import jax
import jax.numpy as jnp
from jax.experimental import pallas as pl


def kernel(x_node, x_trace, x_log, node_adj, edge_adj, edge_efea, W_l1, b_l1, W_r1, b_r1, W_e1, att1, bias1, W_l2, b_l2, W_r2, b_r2, W_e2, att2, bias2):
    raise NotImplementedError("write your pallas kernel here")


# speedup vs baseline: 11.1816x; 11.1816x over previous
import functools

import jax
import jax.numpy as jnp
from jax import lax
from jax.experimental import pallas as pl
from jax.experimental.pallas import tpu as pltpu

H1, H2 = 4, 4
B, W, NN, NT = 8, 16, 200, 1600
ND, ED, LD = 128, 32, 64
D = ND + LD
N = B * W * NN
T = B * W * NT
C1 = D // H1
C2 = ED // H2


def _mm2_body(x_ref, wa_ref, ba_ref, wb_ref, bb_ref, oa_ref, ob_ref):
    x = x_ref[...]
    oa_ref[...] = jnp.dot(x, wa_ref[...], preferred_element_type=jnp.float32) + ba_ref[...]
    ob_ref[...] = jnp.dot(x, wb_ref[...], preferred_element_type=jnp.float32) + bb_ref[...]


def _mm2(x, wa, ba, wb, bb, br):
    n, k = x.shape
    m = wa.shape[1]
    grid = (n // br,)
    return pl.pallas_call(
        _mm2_body,
        grid=grid,
        in_specs=[
            pl.BlockSpec((br, k), lambda i: (i, 0)),
            pl.BlockSpec((k, m), lambda i: (0, 0)),
            pl.BlockSpec((1, m), lambda i: (0, 0)),
            pl.BlockSpec((k, m), lambda i: (0, 0)),
            pl.BlockSpec((1, m), lambda i: (0, 0)),
        ],
        out_specs=[
            pl.BlockSpec((br, m), lambda i: (i, 0)),
            pl.BlockSpec((br, m), lambda i: (i, 0)),
        ],
        out_shape=[
            jax.ShapeDtypeStruct((n, m), jnp.float32),
            jax.ShapeDtypeStruct((n, m), jnp.float32),
        ],
    )(x, wa, ba.reshape(1, m), wb, bb.reshape(1, m))


def _mm1_body(x_ref, w_ref, o_ref):
    o_ref[...] = jnp.dot(x_ref[...], w_ref[...], preferred_element_type=jnp.float32)


def _mm1(x, w, br):
    n, k = x.shape
    m = w.shape[1]
    return pl.pallas_call(
        _mm1_body,
        grid=(n // br,),
        in_specs=[
            pl.BlockSpec((br, k), lambda i: (i, 0)),
            pl.BlockSpec((k, m), lambda i: (0, 0)),
        ],
        out_specs=pl.BlockSpec((br, m), lambda i: (i, 0)),
        out_shape=jax.ShapeDtypeStruct((n, m), jnp.float32),
    )(x, w)


def _epass_body(gs_ref, gd_ref, xe_ref, att_ref, seg_ref, sel_ref,
                v_ref, ex_ref):
    e = gs_ref[...] + gd_ref[...] + xe_ref[...]
    e = jnp.where(e >= 0, e, 0.2 * e)
    alpha = jnp.dot(e * att_ref[...], seg_ref[...],
                    preferred_element_type=jnp.float32)
    ex = jnp.exp(alpha)
    exp16 = jnp.pad(ex, ((0, 0), (0, 16 - ex.shape[1])))
    ex_ref[...] = exp16
    v_ref[...] = gs_ref[...] * jnp.dot(ex, sel_ref[...],
                                       preferred_element_type=jnp.float32)


def _epass(gs, gd, xe, att_flat, seg, sel, br):
    t, d = gs.shape
    h = seg.shape[1]
    return pl.pallas_call(
        _epass_body,
        grid=(t // br,),
        in_specs=[
            pl.BlockSpec((br, d), lambda i: (i, 0)),
            pl.BlockSpec((br, d), lambda i: (i, 0)),
            pl.BlockSpec((br, d), lambda i: (i, 0)),
            pl.BlockSpec((1, d), lambda i: (0, 0)),
            pl.BlockSpec((d, h), lambda i: (0, 0)),
            pl.BlockSpec((h, d), lambda i: (0, 0)),
        ],
        out_specs=[
            pl.BlockSpec((br, d), lambda i: (i, 0)),
            pl.BlockSpec((br, 16), lambda i: (i, 0)),
        ],
        out_shape=[
            jax.ShapeDtypeStruct((t, d), jnp.float32),
            jax.ShapeDtypeStruct((t, 16), jnp.float32),
        ],
    )(gs, gd, xe, att_flat, seg, sel)


def _post_body(acc_ref, den_ref, sel_ref, bias_ref, o_ref):
    den = den_ref[...][:, : sel_ref.shape[0]]
    deng = jnp.dot(den, sel_ref[...], preferred_element_type=jnp.float32)
    o_ref[...] = acc_ref[...] / (deng + 1e-16) + bias_ref[...]


def _post(acc, den, sel, bias, br):
    n, d = acc.shape
    h = sel.shape[0]
    return pl.pallas_call(
        _post_body,
        grid=(n // br,),
        in_specs=[
            pl.BlockSpec((br, d), lambda i: (i, 0)),
            pl.BlockSpec((br, 16), lambda i: (i, 0)),
            pl.BlockSpec((h, d), lambda i: (0, 0)),
            pl.BlockSpec((1, d), lambda i: (0, 0)),
        ],
        out_specs=pl.BlockSpec((br, d), lambda i: (i, 0)),
        out_shape=jax.ShapeDtypeStruct((n, d), jnp.float32),
    )(acc, den, sel, bias.reshape(1, d))


def _head_sel(h, c):
    return (jnp.arange(h * c)[None, :] // c == jnp.arange(h)[:, None]).astype(jnp.float32)


def kernel(x_node, x_trace, x_log, node_adj, edge_adj, edge_efea,
           W_l1, b_l1, W_r1, b_r1, W_e1, att1, bias1,
           W_l2, b_l2, W_r2, b_r2, W_e2, att2, bias2):
    node = jnp.concatenate([x_node, x_log], axis=-1).reshape(-1, D)
    tr = x_trace.reshape(-1, ED)
    src1, dst1 = node_adj[0], node_adj[1]
    src2, dst2 = edge_adj[0], edge_adj[1]

    sel1 = _head_sel(H1, C1)
    sel2 = _head_sel(H2, C2)
    att1f = att1.reshape(1, D)
    att2f = att2.reshape(1, ED)

    xl1, xr1 = _mm2(node, W_l1, b_l1, W_r1, b_r1, br=1600)
    xe1 = _mm1(tr, W_e1, br=4096)

    gs1 = jnp.take(xl1, src1, axis=0)
    gd1 = jnp.take(xr1, dst1, axis=0)
    v1, ex1 = _epass(gs1, gd1, xe1, att1f, sel1.T, sel1, br=2048)
    acc1 = jax.ops.segment_sum(v1, dst1, num_segments=N)
    den1 = jax.ops.segment_sum(ex1, dst1, num_segments=N)
    node_out = _post(acc1, den1, sel1, bias1, br=1600)

    xl2, xr2 = _mm2(tr, W_l2, b_l2, W_r2, b_r2, br=4096)
    ge = jnp.take(node_out, edge_efea, axis=0)
    xe2 = _mm1(ge, W_e2, br=4096)

    gs2 = jnp.take(xl2, src2, axis=0)
    gd2 = jnp.take(xr2, dst2, axis=0)
    v2, ex2 = _epass(gs2, gd2, xe2, att2f, sel2.T, sel2, br=4096)
    acc2 = jax.ops.segment_sum(v2, dst2, num_segments=T)
    den2 = jax.ops.segment_sum(ex2, dst2, num_segments=T)
    tr_out = _post(acc2, den2, sel2, bias2, br=4096)

    xn = node_out[:, :ND].reshape(B, W, NN, ND)
    xt = tr_out.reshape(B, W, NT, ED)
    xl = node_out[:, ND:].reshape(B, W, NN, LD)
    return (xn, xt, xl)


# speedup vs baseline: 13.1168x; 1.1731x over previous
import functools

import jax
import jax.numpy as jnp
from jax import lax
from jax.experimental import pallas as pl
from jax.experimental.pallas import tpu as pltpu
from jax.experimental.pallas import tpu_sc as plsc

_SC_CORES, _SC_SUBCORES = 2, 16
_SC_WORKERS = _SC_CORES * _SC_SUBCORES

H1, H2 = 4, 4
B, W, NN, NT = 8, 16, 200, 1600
ND, ED, LD = 128, 32, 64
D = ND + LD
N = B * W * NN
T = B * W * NT
C1 = D // H1
C2 = ED // H2


def _mm2_body(x_ref, wa_ref, ba_ref, wb_ref, bb_ref, oa_ref, ob_ref):
    x = x_ref[...]
    m = wa_ref.shape[1]
    ow = oa_ref.shape[1]
    ya = jnp.dot(x, wa_ref[...], preferred_element_type=jnp.float32) + ba_ref[...]
    yb = jnp.dot(x, wb_ref[...], preferred_element_type=jnp.float32) + bb_ref[...]
    oa_ref[...] = jnp.pad(ya, ((0, 0), (0, ow - m)))
    ob_ref[...] = jnp.pad(yb, ((0, 0), (0, ow - m)))


def _mm2(x, wa, ba, wb, bb, br, ow=None):
    n, k = x.shape
    m = wa.shape[1]
    ow = m if ow is None else ow
    grid = (n // br,)
    return pl.pallas_call(
        _mm2_body,
        grid=grid,
        in_specs=[
            pl.BlockSpec((br, k), lambda i: (i, 0)),
            pl.BlockSpec((k, m), lambda i: (0, 0)),
            pl.BlockSpec((1, m), lambda i: (0, 0)),
            pl.BlockSpec((k, m), lambda i: (0, 0)),
            pl.BlockSpec((1, m), lambda i: (0, 0)),
        ],
        out_specs=[
            pl.BlockSpec((br, ow), lambda i: (i, 0)),
            pl.BlockSpec((br, ow), lambda i: (i, 0)),
        ],
        out_shape=[
            jax.ShapeDtypeStruct((n, ow), jnp.float32),
            jax.ShapeDtypeStruct((n, ow), jnp.float32),
        ],
    )(x, wa, ba.reshape(1, m), wb, bb.reshape(1, m))


def _mm1_body(x_ref, w_ref, o_ref):
    k = w_ref.shape[0]
    o_ref[...] = jnp.dot(x_ref[...][:, :k], w_ref[...],
                         preferred_element_type=jnp.float32)


def _mm1(x, w, br):
    n, kp = x.shape
    k, m = w.shape
    return pl.pallas_call(
        _mm1_body,
        grid=(n // br,),
        in_specs=[
            pl.BlockSpec((br, kp), lambda i: (i, 0)),
            pl.BlockSpec((k, m), lambda i: (0, 0)),
        ],
        out_specs=pl.BlockSpec((br, m), lambda i: (i, 0)),
        out_shape=jax.ShapeDtypeStruct((n, m), jnp.float32),
    )(x, w)


def _epass_body(gs_ref, gd_ref, xe_ref, att_ref, seg_ref, sel_ref,
                *out_refs):
    d = xe_ref.shape[1]
    ex_ref = out_refs[-1]
    v_refs = out_refs[:-1]
    vw = v_refs[0].shape[1]
    gs = gs_ref[...][:, :d]
    e = gs + gd_ref[...][:, :d] + xe_ref[...]
    e = jnp.where(e >= 0, e, 0.2 * e)
    alpha = jnp.dot(e * att_ref[...], seg_ref[...],
                    preferred_element_type=jnp.float32)
    ex = jnp.exp(alpha)
    ex_ref[...] = jnp.pad(ex, ((0, 0), (0, ex_ref.shape[1] - ex.shape[1])))
    v = gs * jnp.dot(ex, sel_ref[...], preferred_element_type=jnp.float32)
    for j, vr in enumerate(v_refs):
        vr[...] = v[:, j * vw:(j + 1) * vw]


def _epass(gs, gd, xe, att_flat, seg, sel, br, vw, ew=16):
    t, dp = gs.shape
    d = xe.shape[1]
    h = seg.shape[1]
    nv = d // vw
    outs = pl.pallas_call(
        _epass_body,
        grid=(t // br,),
        in_specs=[
            pl.BlockSpec((br, dp), lambda i: (i, 0)),
            pl.BlockSpec((br, dp), lambda i: (i, 0)),
            pl.BlockSpec((br, d), lambda i: (i, 0)),
            pl.BlockSpec((1, d), lambda i: (0, 0)),
            pl.BlockSpec((d, h), lambda i: (0, 0)),
            pl.BlockSpec((h, d), lambda i: (0, 0)),
        ],
        out_specs=[pl.BlockSpec((br, vw), lambda i: (i, 0))
                   for _ in range(nv)]
        + [pl.BlockSpec((br, ew), lambda i: (i, 0))],
        out_shape=[jax.ShapeDtypeStruct((t, vw), jnp.float32)
                   for _ in range(nv)]
        + [jax.ShapeDtypeStruct((t, ew), jnp.float32)],
    )(gs, gd, xe, att_flat, seg, sel)
    return outs[:-1], outs[-1]


def _post(accs, dens, sel, bias, br, ow=None):
    n = accs[0].shape[0]
    cw = accs[0].shape[1]
    d = cw * len(accs)
    h = sel.shape[0]
    ow = d if ow is None else ow
    na, nd = len(accs), len(dens)

    def body(*refs):
        acc_refs = refs[:na]
        den_refs = refs[na:na + nd]
        sel_ref, bias_ref = refs[na + nd], refs[na + nd + 1]
        o_ref = refs[na + nd + 2]
        den = den_refs[0][...]
        for dref in den_refs[1:]:
            den = den + dref[...]
        deng = jnp.dot(den[:, :h], sel_ref[...],
                       preferred_element_type=jnp.float32)
        acc = jnp.concatenate([aref[...] for aref in acc_refs], axis=1)
        y = acc / (deng + 1e-16) + bias_ref[...]
        o_ref[...] = jnp.pad(y, ((0, 0), (0, ow - d)))

    return pl.pallas_call(
        body,
        grid=(n // br,),
        in_specs=[pl.BlockSpec((br, cw), lambda i: (i, 0))
                  for _ in range(na)]
        + [pl.BlockSpec((br, dens[0].shape[1]), lambda i: (i, 0))
           for _ in range(nd)]
        + [
            pl.BlockSpec((h, d), lambda i: (0, 0)),
            pl.BlockSpec((1, d), lambda i: (0, 0)),
        ],
        out_specs=pl.BlockSpec((br, ow), lambda i: (i, 0)),
        out_shape=jax.ShapeDtypeStruct((n, ow), jnp.float32),
    )(*accs, *dens, sel, bias.reshape(1, d))


def _head_sel(h, c):
    return (jnp.arange(h * c)[None, :] // c == jnp.arange(h)[:, None]).astype(jnp.float32)


def _sc_gather(tables, idxs, win=256):
    k = len(tables)
    tn = idxs[0].shape[0]
    bpw = tn // _SC_WORKERS
    n_win = bpw // win
    mesh = plsc.VectorSubcoreMesh(
        core_axis_name="c", subcore_axis_name="s",
        num_cores=_SC_CORES, num_subcores=_SC_SUBCORES)
    out_type = [jax.ShapeDtypeStruct((tn, t.shape[1]), jnp.float32)
                for t in tables]
    scratch = []
    for t in tables:
        scratch += [pltpu.VMEM((win,), jnp.int32),
                    pltpu.VMEM((win, t.shape[1]), jnp.float32),
                    pltpu.SemaphoreType.DMA]

    def body(*refs):
        tab_refs = refs[:k]
        idx_refs = refs[k:2 * k]
        out_refs = refs[2 * k:3 * k]
        sc = refs[3 * k:]
        wid = lax.axis_index("s") * _SC_CORES + lax.axis_index("c")
        base = wid * bpw

        def step(w, carry):
            start = base + w * win
            for j in range(k):
                iv, rv, sem = sc[3 * j], sc[3 * j + 1], sc[3 * j + 2]
                pltpu.sync_copy(idx_refs[j].at[pl.ds(start, win)], iv)
                pltpu.async_copy(tab_refs[j].at[iv], rv, sem).wait()
                pltpu.sync_copy(rv, out_refs[j].at[pl.ds(start, win)])
            return carry

        lax.fori_loop(0, n_win, step, 0)

    fn = pl.kernel(body, out_type=out_type, mesh=mesh, scratch_types=scratch)
    return fn(*tables, *idxs)


def _zero_fill(buf):
    rows, cols = buf.shape
    z = jnp.zeros((16,), jnp.float32)

    def zrow(r, carry):
        for c in range(cols // 16):
            buf[r, pl.ds(c * 16, 16)] = z
        return carry

    lax.fori_loop(0, rows, zrow, 0)


def _sc_scatter1(v_chunks, exw, dst, n_seg):
    tn = dst.shape[0]
    epw = tn // _SC_WORKERS
    ept = tn // _SC_SUBCORES
    win = 320
    nrows = n_seg // _SC_SUBCORES
    zr = nrows // 8
    nc = len(v_chunks)
    mesh = plsc.VectorSubcoreMesh(
        core_axis_name="c", subcore_axis_name="s",
        num_cores=_SC_CORES, num_subcores=_SC_SUBCORES)
    out_type = ([jax.ShapeDtypeStruct((n_seg, 16), jnp.float32)
                 for _ in range(nc)]
                + [jax.ShapeDtypeStruct((2, n_seg, 16), jnp.float32)])
    scratch = [
        pltpu.VMEM_SHARED((n_seg, 16), jnp.float32),
        pltpu.VMEM((zr, 16), jnp.float32),
        pltpu.VMEM((zr, 16), jnp.float32),
        pltpu.VMEM((win,), jnp.int32),
        pltpu.VMEM((win, 16), jnp.float32),
        pltpu.SemaphoreType.DMA,
    ]

    def body(*refs):
        vc = refs[0:nc]
        ex_ref, dst_ref = refs[nc], refs[nc + 1]
        acc_out = refs[nc + 2:2 * nc + 2]
        den_out = refs[2 * nc + 2]
        acc_sh, zbuf, bounce, dst_v, val_v, sem = refs[2 * nc + 3:]
        c = lax.axis_index("c")
        s = lax.axis_index("s")
        _zero_fill(zbuf)

        def run_pass(val_ref, base, n_win, out_ref, out_row0):
            for p in range(8):
                pltpu.sync_copy(zbuf, acc_sh.at[pl.ds(s * nrows + p * zr, zr)])
            plsc.subcore_barrier()

            def step(w, carry):
                start = base + w * win
                pltpu.sync_copy(dst_ref.at[pl.ds(start, win)], dst_v)
                pltpu.sync_copy(val_ref.at[pl.ds(start, win)], val_v)
                pltpu.sync_copy(val_v, acc_sh.at[dst_v], add=True)
                return carry

            lax.fori_loop(0, n_win, step, 0)
            plsc.subcore_barrier()
            for p in range(8):
                r0 = s * nrows + p * zr
                pltpu.sync_copy(acc_sh.at[pl.ds(r0, zr)], bounce)
                if out_row0 is None:
                    pltpu.sync_copy(bounce, out_ref.at[pl.ds(r0, zr)])
                else:
                    pltpu.sync_copy(bounce,
                                    out_ref.at[out_row0, pl.ds(r0, zr)])
            plsc.subcore_barrier()

        for j in range(nc):
            @pl.when(c == j // (nc // 2))
            def _chunk(j=j):
                run_pass(vc[j], s * ept, ept // win, acc_out[j], None)

        run_pass(ex_ref, (c * _SC_SUBCORES + s) * epw, epw // win,
                 den_out, c)

    fn = pl.kernel(body, out_type=out_type, mesh=mesh, scratch_types=scratch)
    return fn(*v_chunks, exw, dst)


def _sc_scatter2(vals, dst, n_seg):
    tn = dst.shape[0]
    ept = tn // _SC_SUBCORES
    win = 320
    nrng = 8
    qtr = n_seg // nrng
    dum = 512
    brows = qtr + dum
    nrows = brows // _SC_SUBCORES
    drows = qtr // _SC_SUBCORES
    zr = nrows // 8
    dr = drows // 8
    mesh = plsc.VectorSubcoreMesh(
        core_axis_name="c", subcore_axis_name="s",
        num_cores=_SC_CORES, num_subcores=_SC_SUBCORES)
    out_type = [jax.ShapeDtypeStruct((n_seg, 16), jnp.float32)
                for _ in range(3)]
    scratch = [
        pltpu.VMEM_SHARED((brows, 16), jnp.float32),
        pltpu.VMEM((zr, 16), jnp.float32),
        pltpu.VMEM((dr, 16), jnp.float32),
        pltpu.VMEM((win,), jnp.int32),
        pltpu.VMEM((win,), jnp.int32),
        pltpu.VMEM((win, 16), jnp.float32),
        pltpu.SemaphoreType.DMA,
    ]

    def body(*refs):
        vin = refs[0:3]
        dst_ref = refs[3]
        vout = refs[4:7]
        buf_sh, zbuf, bounce, dst_v, idx_v, val_v, sem = refs[7:]
        c = lax.axis_index("c")
        s = lax.axis_index("s")
        _zero_fill(zbuf)
        lanes = lax.iota(jnp.int32, 16)

        for job in range(3 * nrng):
            a = job // nrng
            r = job % nrng

            @pl.when(c == job // (3 * nrng // 2))
            def _job(a=a, r=r):
                for p in range(8):
                    pltpu.sync_copy(
                        zbuf, buf_sh.at[pl.ds(s * nrows + p * zr, zr)])
                plsc.subcore_barrier()
                base = s * ept
                lo = r * qtr

                def step(w, carry):
                    start = base + w * win
                    pltpu.sync_copy(dst_ref.at[pl.ds(start, win)], dst_v)

                    def remap(k, carry2):
                        v = dst_v[pl.ds(k * 16, 16)] - lo
                        ok = (v >= 0) & (v < qtr)
                        dmy = qtr + lax.rem(w * 11 + k, 31) * 16 + lanes
                        idx_v[pl.ds(k * 16, 16)] = jnp.where(ok, v, dmy)
                        return carry2

                    lax.fori_loop(0, win // 16, remap, 0)
                    pltpu.sync_copy(vin[a].at[pl.ds(start, win)], val_v)
                    pltpu.sync_copy(val_v, buf_sh.at[idx_v], add=True)
                    return carry

                lax.fori_loop(0, ept // win, step, 0)
                plsc.subcore_barrier()
                for p in range(8):
                    pltpu.sync_copy(buf_sh.at[pl.ds(s * drows + p * dr, dr)],
                                    bounce)
                    pltpu.sync_copy(
                        bounce,
                        vout[a].at[pl.ds(lo + s * drows + p * dr, dr)])
                plsc.subcore_barrier()

    fn = pl.kernel(body, out_type=out_type, mesh=mesh, scratch_types=scratch)
    return fn(*vals, dst)


def kernel(x_node, x_trace, x_log, node_adj, edge_adj, edge_efea,
           W_l1, b_l1, W_r1, b_r1, W_e1, att1, bias1,
           W_l2, b_l2, W_r2, b_r2, W_e2, att2, bias2):
    node = jnp.concatenate([x_node, x_log], axis=-1).reshape(-1, D)
    tr = x_trace.reshape(-1, ED)
    src1, dst1 = node_adj[0], node_adj[1]
    src2, dst2 = edge_adj[0], edge_adj[1]

    sel1 = _head_sel(H1, C1)
    sel2 = _head_sel(H2, C2)
    att1f = att1.reshape(1, D)
    att2f = att2.reshape(1, ED)

    xl1, xr1 = _mm2(node, W_l1, b_l1, W_r1, b_r1, br=1600, ow=256)
    xe1 = _mm1(tr, W_e1, br=4096)

    gs1, gd1 = _sc_gather([xl1, xr1], [src1, dst1], win=64)
    v1c, ex1 = _epass(gs1, gd1, xe1, att1f, sel1.T, sel1, br=2048, vw=16,
                      ew=16)
    acc1 = jax.ops.segment_sum(jnp.concatenate(v1c, axis=1), dst1,
                               num_segments=N)
    den1 = jax.ops.segment_sum(ex1, dst1, num_segments=N)
    node_out = _post([acc1], [den1], sel1, bias1, br=1600, ow=256)

    xl2, xr2 = _mm2(tr, W_l2, b_l2, W_r2, b_r2, br=4096, ow=128)
    gs2, gd2, ge = _sc_gather([xl2, xr2, node_out], [src2, dst2, edge_efea],
                              win=64)
    xe2 = _mm1(ge, W_e2, br=4096)
    v2c, ex2 = _epass(gs2, gd2, xe2, att2f, sel2.T, sel2, br=4096, vw=16,
                      ew=16)
    acc2 = jax.ops.segment_sum(jnp.concatenate(v2c, axis=1), dst2,
                               num_segments=T)
    den2 = jax.ops.segment_sum(ex2, dst2, num_segments=T)
    tr_out = _post([acc2], [den2], sel2, bias2, br=4096)

    xn = node_out[:, :ND].reshape(B, W, NN, ND)
    xt = tr_out.reshape(B, W, NT, ED)
    xl = node_out[:, ND:D].reshape(B, W, NN, LD)
    return (xn, xt, xl)


# speedup vs baseline: 17.0538x; 1.3002x over previous
import functools

import jax
import jax.numpy as jnp
from jax import lax
from jax.experimental import pallas as pl
from jax.experimental.pallas import tpu as pltpu
from jax.experimental.pallas import tpu_sc as plsc

_SC_CORES, _SC_SUBCORES = 2, 16
_SC_WORKERS = _SC_CORES * _SC_SUBCORES

H1, H2 = 4, 4
B, W, NN, NT = 8, 16, 200, 1600
ND, ED, LD = 128, 32, 64
D = ND + LD
N = B * W * NN
T = B * W * NT
C1 = D // H1
C2 = ED // H2


def _mm2_body(x_ref, wa_ref, ba_ref, wb_ref, bb_ref, oa_ref, ob_ref):
    x = x_ref[...]
    m = wa_ref.shape[1]
    ow = oa_ref.shape[1]
    ya = jnp.dot(x, wa_ref[...], preferred_element_type=jnp.float32) + ba_ref[...]
    yb = jnp.dot(x, wb_ref[...], preferred_element_type=jnp.float32) + bb_ref[...]
    oa_ref[...] = jnp.pad(ya, ((0, 0), (0, ow - m)))
    ob_ref[...] = jnp.pad(yb, ((0, 0), (0, ow - m)))


def _mm2(x, wa, ba, wb, bb, br, ow=None):
    n, k = x.shape
    m = wa.shape[1]
    ow = m if ow is None else ow
    grid = (n // br,)
    return pl.pallas_call(
        _mm2_body,
        grid=grid,
        in_specs=[
            pl.BlockSpec((br, k), lambda i: (i, 0)),
            pl.BlockSpec((k, m), lambda i: (0, 0)),
            pl.BlockSpec((1, m), lambda i: (0, 0)),
            pl.BlockSpec((k, m), lambda i: (0, 0)),
            pl.BlockSpec((1, m), lambda i: (0, 0)),
        ],
        out_specs=[
            pl.BlockSpec((br, ow), lambda i: (i, 0)),
            pl.BlockSpec((br, ow), lambda i: (i, 0)),
        ],
        out_shape=[
            jax.ShapeDtypeStruct((n, ow), jnp.float32),
            jax.ShapeDtypeStruct((n, ow), jnp.float32),
        ],
    )(x, wa, ba.reshape(1, m), wb, bb.reshape(1, m))


def _mm1_body(x_ref, w_ref, o_ref):
    k = w_ref.shape[0]
    o_ref[...] = jnp.dot(x_ref[...][:, :k], w_ref[...],
                         preferred_element_type=jnp.float32)


def _mm1(x, w, br):
    n, kp = x.shape
    k, m = w.shape
    return pl.pallas_call(
        _mm1_body,
        grid=(n // br,),
        in_specs=[
            pl.BlockSpec((br, kp), lambda i: (i, 0)),
            pl.BlockSpec((k, m), lambda i: (0, 0)),
        ],
        out_specs=pl.BlockSpec((br, m), lambda i: (i, 0)),
        out_shape=jax.ShapeDtypeStruct((n, m), jnp.float32),
    )(x, w)


def _epass_body(gs_ref, gd_ref, xe_ref, att_ref, seg_ref, sel_ref,
                *out_refs):
    d = xe_ref.shape[1]
    ex_ref = out_refs[-1]
    v_refs = out_refs[:-1]
    vw = v_refs[0].shape[1]
    gs = gs_ref[...][:, :d]
    e = gs + gd_ref[...][:, :d] + xe_ref[...]
    e = jnp.where(e >= 0, e, 0.2 * e)
    alpha = jnp.dot(e * att_ref[...], seg_ref[...],
                    preferred_element_type=jnp.float32)
    ex = jnp.exp(alpha)
    ex_ref[...] = jnp.pad(ex, ((0, 0), (0, ex_ref.shape[1] - ex.shape[1])))
    v = gs * jnp.dot(ex, sel_ref[...], preferred_element_type=jnp.float32)
    for j, vr in enumerate(v_refs):
        vr[...] = v[:, j * vw:(j + 1) * vw]


def _epass(gs, gd, xe, att_flat, seg, sel, br, vw, ew=16):
    t, dp = gs.shape
    d = xe.shape[1]
    h = seg.shape[1]
    nv = d // vw
    outs = pl.pallas_call(
        _epass_body,
        grid=(t // br,),
        in_specs=[
            pl.BlockSpec((br, dp), lambda i: (i, 0)),
            pl.BlockSpec((br, dp), lambda i: (i, 0)),
            pl.BlockSpec((br, d), lambda i: (i, 0)),
            pl.BlockSpec((1, d), lambda i: (0, 0)),
            pl.BlockSpec((d, h), lambda i: (0, 0)),
            pl.BlockSpec((h, d), lambda i: (0, 0)),
        ],
        out_specs=[pl.BlockSpec((br, vw), lambda i: (i, 0))
                   for _ in range(nv)]
        + [pl.BlockSpec((br, ew), lambda i: (i, 0))],
        out_shape=[jax.ShapeDtypeStruct((t, vw), jnp.float32)
                   for _ in range(nv)]
        + [jax.ShapeDtypeStruct((t, ew), jnp.float32)],
    )(gs, gd, xe, att_flat, seg, sel)
    return outs[:-1], outs[-1]


def _post(accs, dens, sel, bias, br, ow=None):
    n = accs[0].shape[0]
    cw = accs[0].shape[1]
    d = cw * len(accs)
    h = sel.shape[0]
    ow = d if ow is None else ow
    na, nd = len(accs), len(dens)

    def body(*refs):
        acc_refs = refs[:na]
        den_refs = refs[na:na + nd]
        sel_ref, bias_ref = refs[na + nd], refs[na + nd + 1]
        o_ref = refs[na + nd + 2]
        den = den_refs[0][...]
        for dref in den_refs[1:]:
            den = den + dref[...]
        deng = jnp.dot(den[:, :h], sel_ref[...],
                       preferred_element_type=jnp.float32)
        acc = jnp.concatenate([aref[...] for aref in acc_refs], axis=1)
        y = acc / (deng + 1e-16) + bias_ref[...]
        o_ref[...] = jnp.pad(y, ((0, 0), (0, ow - d)))

    return pl.pallas_call(
        body,
        grid=(n // br,),
        in_specs=[pl.BlockSpec((br, cw), lambda i: (i, 0))
                  for _ in range(na)]
        + [pl.BlockSpec((br, dens[0].shape[1]), lambda i: (i, 0))
           for _ in range(nd)]
        + [
            pl.BlockSpec((h, d), lambda i: (0, 0)),
            pl.BlockSpec((1, d), lambda i: (0, 0)),
        ],
        out_specs=pl.BlockSpec((br, ow), lambda i: (i, 0)),
        out_shape=jax.ShapeDtypeStruct((n, ow), jnp.float32),
    )(*accs, *dens, sel, bias.reshape(1, d))


def _head_sel(h, c):
    return (jnp.arange(h * c)[None, :] // c == jnp.arange(h)[:, None]).astype(jnp.float32)


def _sc_gather(tables, idxs, win=256):
    k = len(tables)
    tn = idxs[0].shape[0]
    bpw = tn // _SC_WORKERS
    n_win = bpw // win
    mesh = plsc.VectorSubcoreMesh(
        core_axis_name="c", subcore_axis_name="s",
        num_cores=_SC_CORES, num_subcores=_SC_SUBCORES)
    out_type = [jax.ShapeDtypeStruct((tn, t.shape[1]), jnp.float32)
                for t in tables]
    scratch = []
    for t in tables:
        scratch += [pltpu.VMEM((win,), jnp.int32),
                    pltpu.VMEM((win, t.shape[1]), jnp.float32),
                    pltpu.SemaphoreType.DMA]

    def body(*refs):
        tab_refs = refs[:k]
        idx_refs = refs[k:2 * k]
        out_refs = refs[2 * k:3 * k]
        sc = refs[3 * k:]
        wid = lax.axis_index("s") * _SC_CORES + lax.axis_index("c")
        base = wid * bpw

        @pl.loop(0, n_win)
        def step(w):
            start = base + w * win
            for j in range(k):
                iv, rv, sem = sc[3 * j], sc[3 * j + 1], sc[3 * j + 2]
                pltpu.sync_copy(idx_refs[j].at[pl.ds(start, win)], iv)
                pltpu.async_copy(tab_refs[j].at[iv], rv, sem).wait()
                pltpu.sync_copy(rv, out_refs[j].at[pl.ds(start, win)])

    fn = pl.kernel(body, out_type=out_type, mesh=mesh, scratch_types=scratch)
    return fn(*tables, *idxs)


def kernel(x_node, x_trace, x_log, node_adj, edge_adj, edge_efea,
           W_l1, b_l1, W_r1, b_r1, W_e1, att1, bias1,
           W_l2, b_l2, W_r2, b_r2, W_e2, att2, bias2):
    node = jnp.concatenate([x_node, x_log], axis=-1).reshape(-1, D)
    tr = x_trace.reshape(-1, ED)
    src1, dst1 = node_adj[0], node_adj[1]
    src2, dst2 = edge_adj[0], edge_adj[1]

    sel1 = _head_sel(H1, C1)
    sel2 = _head_sel(H2, C2)
    att1f = att1.reshape(1, D)
    att2f = att2.reshape(1, ED)

    xl1, xr1 = _mm2(node, W_l1, b_l1, W_r1, b_r1, br=1600, ow=256)
    xe1 = _mm1(tr, W_e1, br=4096)

    gs1, gd1 = _sc_gather([xl1, xr1], [src1, dst1], win=80)
    v1c, ex1 = _epass(gs1, gd1, xe1, att1f, sel1.T, sel1, br=2048, vw=192,
                      ew=16)
    acc1 = jax.ops.segment_sum(v1c[0], dst1, num_segments=N)
    den1 = jax.ops.segment_sum(ex1, dst1, num_segments=N)
    node_out = _post([acc1], [den1], sel1, bias1, br=1600, ow=256)

    xl2, xr2 = _mm2(tr, W_l2, b_l2, W_r2, b_r2, br=4096, ow=128)
    gs2, gd2, ge = _sc_gather([xl2, xr2, node_out], [src2, dst2, edge_efea],
                              win=80)
    xe2 = _mm1(ge, W_e2, br=4096)
    v2c, ex2 = _epass(gs2, gd2, xe2, att2f, sel2.T, sel2, br=4096, vw=32,
                      ew=16)
    acc2 = jax.ops.segment_sum(v2c[0], dst2, num_segments=T)
    den2 = jax.ops.segment_sum(ex2, dst2, num_segments=T)
    tr_out = _post([acc2], [den2], sel2, bias2, br=4096)

    xn = node_out[:, :ND].reshape(B, W, NN, ND)
    xt = tr_out.reshape(B, W, NT, ED)
    xl = node_out[:, ND:D].reshape(B, W, NN, LD)
    return (xn, xt, xl)
